# baseline (speedup 1.0000x reference)
import jax
import jax.numpy as jnp
from jax.experimental import pallas as pl

_Q = 16384
_K = 256
_D = 128
_TQ = 8192


def _head_kernel(q_ref, s_ref, t_ref, correct_ref, loss_ref):
    i = pl.program_id(0)
    g = pl.num_programs(0)
    q = q_ref[...]
    s = s_ref[...]
    scores = jax.lax.dot_general(
        s, q, (((1,), (1,)), ((), ())), preferred_element_type=jnp.float32
    )
    t = t_ref[0, :, :]
    iota = jax.lax.broadcasted_iota(jnp.int32, scores.shape, 0)
    m = jnp.max(scores, axis=0, keepdims=True)
    tgt = jnp.sum(jnp.where(iota == t, scores, 0.0), axis=0, keepdims=True)
    preds = jnp.min(jnp.where(scores == m, iota, _K), axis=0, keepdims=True)
    correct_ref[0, :, :] = (preds == t).astype(jnp.int8)
    lse = m + jnp.log(jnp.sum(jnp.exp(scores - m), axis=0, keepdims=True))
    nll_sum = jnp.sum(lse - tgt)
    prev = jnp.where(i == 0, jnp.zeros_like(loss_ref[...]), loss_ref[...])
    acc = prev + nll_sum
    loss_ref[...] = jnp.where(i == g - 1, acc / _Q, acc)


@jax.jit
def kernel(query_reps, support_reps, target_ids):
    grid = _Q // _TQ
    targets = target_ids.astype(jnp.int32).reshape(grid, 1, _TQ)
    correct8, loss = pl.pallas_call(
        _head_kernel,
        grid=(grid,),
        in_specs=[
            pl.BlockSpec((_TQ, _D), lambda i: (i, 0)),
            pl.BlockSpec((_K, _D), lambda i: (0, 0)),
            pl.BlockSpec((1, 1, _TQ), lambda i: (i, 0, 0)),
        ],
        out_specs=[
            pl.BlockSpec((1, 1, _TQ), lambda i: (i, 0, 0)),
            pl.BlockSpec((1, 128), lambda i: (0, 0)),
        ],
        out_shape=[
            jax.ShapeDtypeStruct((grid, 1, _TQ), jnp.int8),
            jax.ShapeDtypeStruct((1, 128), jnp.float32),
        ],
    )(query_reps, support_reps, targets)
    return (loss[0, 0], correct8.reshape(_Q).astype(jnp.bool_))


# speedup vs baseline: 1.0495x; 1.0495x over previous
import jax
import jax.numpy as jnp
from jax.experimental import pallas as pl

_Q = 16384
_K = 256
_D = 128
_TQ = 4096


def _head_kernel(q_ref, s_ref, t_ref, correct_ref, loss_ref):
    i = pl.program_id(0)
    g = pl.num_programs(0)
    q = q_ref[...]
    s = s_ref[...]
    scores = jax.lax.dot_general(
        s, q, (((1,), (1,)), ((), ())), preferred_element_type=jnp.float32
    )
    t = t_ref[0, :, :]
    iota = jax.lax.broadcasted_iota(jnp.int32, scores.shape, 0)
    m = jnp.max(scores, axis=0, keepdims=True)
    tgt = jnp.sum(jnp.where(iota == t, scores, 0.0), axis=0, keepdims=True)
    preds = jnp.min(jnp.where(scores == m, iota, _K), axis=0, keepdims=True)
    correct_ref[0, :, :] = (preds == t).astype(jnp.int8)
    lse = m + jnp.log(jnp.sum(jnp.exp(scores - m), axis=0, keepdims=True))
    nll_sum = jnp.sum(lse - tgt)
    prev = jnp.where(i == 0, jnp.zeros_like(loss_ref[...]), loss_ref[...])
    acc = prev + nll_sum
    loss_ref[...] = jnp.where(i == g - 1, acc / _Q, acc)


@jax.jit
def kernel(query_reps, support_reps, target_ids):
    grid = _Q // _TQ
    targets = target_ids.astype(jnp.int32).reshape(grid, 1, _TQ)
    correct8, loss = pl.pallas_call(
        _head_kernel,
        grid=(grid,),
        in_specs=[
            pl.BlockSpec((_TQ, _D), lambda i: (i, 0)),
            pl.BlockSpec((_K, _D), lambda i: (0, 0)),
            pl.BlockSpec((1, 1, _TQ), lambda i: (i, 0, 0)),
        ],
        out_specs=[
            pl.BlockSpec((1, 1, _TQ), lambda i: (i, 0, 0)),
            pl.BlockSpec((1, 128), lambda i: (0, 0)),
        ],
        out_shape=[
            jax.ShapeDtypeStruct((grid, 1, _TQ), jnp.int8),
            jax.ShapeDtypeStruct((1, 128), jnp.float32),
        ],
    )(query_reps, support_reps, targets)
    return (loss[0, 0], correct8.reshape(_Q).astype(jnp.bool_))


# speedup vs baseline: 1.0870x; 1.0357x over previous
import jax
import jax.numpy as jnp
from jax.experimental import pallas as pl

_Q = 16384
_K = 256
_D = 128
_TH = 2048
_GRID = _Q // (2 * _TH)


def _half(s, q, t):
    scores = jax.lax.dot_general(
        s, q, (((1,), (1,)), ((), ())), preferred_element_type=jnp.float32
    )
    iota = jax.lax.broadcasted_iota(jnp.int32, scores.shape, 0)
    m = jnp.max(scores, axis=0, keepdims=True)
    tgt = jnp.sum(jnp.where(iota == t, scores, 0.0), axis=0, keepdims=True)
    preds = jnp.min(jnp.where(scores == m, iota, _K), axis=0, keepdims=True)
    correct = (preds == t).astype(jnp.int8)
    lse = m + jnp.log(jnp.sum(jnp.exp(scores - m), axis=0, keepdims=True))
    return correct, jnp.sum(lse - tgt)


def _head_kernel(qa_ref, qb_ref, s_ref, ta_ref, tb_ref,
                 ca_ref, cb_ref, loss_ref):
    i = pl.program_id(0)
    g = pl.num_programs(0)
    s = s_ref[...]
    ca, nll_a = _half(s, qa_ref[...], ta_ref[0, :, :])
    cb, nll_b = _half(s, qb_ref[...], tb_ref[0, :, :])
    ca_ref[0, :, :] = ca
    cb_ref[0, :, :] = cb
    prev = jnp.where(i == 0, jnp.zeros_like(loss_ref[...]), loss_ref[...])
    acc = prev + (nll_a + nll_b)
    loss_ref[...] = jnp.where(i == g - 1, acc / _Q, acc)


@jax.jit
def kernel(query_reps, support_reps, target_ids):
    targets = target_ids.astype(jnp.int32).reshape(2 * _GRID, 1, _TH)
    ca8, cb8, loss = pl.pallas_call(
        _head_kernel,
        grid=(_GRID,),
        in_specs=[
            pl.BlockSpec((_TH, _D), lambda i: (2 * i, 0)),
            pl.BlockSpec((_TH, _D), lambda i: (2 * i + 1, 0)),
            pl.BlockSpec((_K, _D), lambda i: (0, 0)),
            pl.BlockSpec((1, 1, _TH), lambda i: (2 * i, 0, 0)),
            pl.BlockSpec((1, 1, _TH), lambda i: (2 * i + 1, 0, 0)),
        ],
        out_specs=[
            pl.BlockSpec((1, 1, _TH), lambda i: (i, 0, 0)),
            pl.BlockSpec((1, 1, _TH), lambda i: (i, 0, 0)),
            pl.BlockSpec((1, 128), lambda i: (0, 0)),
        ],
        out_shape=[
            jax.ShapeDtypeStruct((_GRID, 1, _TH), jnp.int8),
            jax.ShapeDtypeStruct((_GRID, 1, _TH), jnp.int8),
            jax.ShapeDtypeStruct((1, 128), jnp.float32),
        ],
    )(query_reps, query_reps, support_reps, targets, targets)
    correct = jnp.stack([ca8[:, 0, :], cb8[:, 0, :]], axis=1).reshape(_Q)
    return (loss[0, 0], correct.astype(jnp.bool_))
